# baseline (speedup 1.0000x reference)
import functools

import jax
import jax.numpy as jnp
from jax import lax
from jax.experimental import pallas as pl
from jax.experimental.pallas import tpu as pltpu
from jax.experimental.pallas import tpu_sc as plsc

USED_GRAPHS = ["sl", "ppi", "reactome", "go_f", "go_c", "go_p", "kegg"]
NGG = len(USED_GRAPHS)
N = 10000
NP = 10240
E = 320000
NC, NS = 2, 16
NW = NC * NS
NGNP = NGG * NP

CH = 128
KB = 8
BE = CH * KB
NB1 = 10
EP1 = NW * NB1 * BE
NB2 = NGG * NB1
EDEC = 2 * E
NBD = 20
EPD = NW * NBD * BE

DEC_BLOCK = 2048

_MESH = plsc.VectorSubcoreMesh(core_axis_name="c", subcore_axis_name="s",
                               num_cores=NC, num_subcores=NS)
_SC_PARAMS = pltpu.CompilerParams(use_tc_tiling_on_sc=False)


def _worker_id():
    return lax.axis_index("s") * NC + lax.axis_index("c")


def _load_idx_batch(src2d_ref, row0, dst_buf):
    pltpu.sync_copy(src2d_ref.at[pl.ds(row0, KB)], dst_buf)


def _mp_batch(t_ref, acc, Is, Id, rows, gsems):
    g = [None, None]
    g[0] = pltpu.async_copy(t_ref.at[Is.at[0]], rows[0], gsems[0])
    for i in range(KB):
        b = i % 2
        if i + 1 < KB:
            g[1 - b] = pltpu.async_copy(t_ref.at[Is.at[i + 1]], rows[1 - b],
                                        gsems[1 - b])
        g[b].wait()
        pltpu.sync_copy(rows[b], acc.at[Id.at[i]], add=True)


def _mp_worker_loop(t_ref, s2d_ref, d2d_ref, acc, Is2, Id2, rows, gsems,
                    isems, row0, nb):
    _load_idx_batch(s2d_ref, row0, Is2[0])
    _load_idx_batch(d2d_ref, row0, Id2[0])

    def outer(bb, _):
        for p in (0, 1):
            k = 2 * bb + p
            nxt = row0 + (k + 1) * KB
            li = pltpu.async_copy(s2d_ref.at[pl.ds(nxt, KB)], Is2[1 - p],
                                  isems[0])
            ld = pltpu.async_copy(d2d_ref.at[pl.ds(nxt, KB)], Id2[1 - p],
                                  isems[1])
            _mp_batch(t_ref, acc, Is2[p], Id2[p], rows, gsems)
            li.wait()
            ld.wait()
        return _

    lax.fori_loop(0, nb // 2, outer, 0)


L1_RPT = NP // NS


def _mp1_body(t_ref, s2d_ref, d2d_ref, out_ref, acc, is0, is1, id0, id1,
              rows0, rows1, gsem0, gsem1, isem0, isem1):
    cid = lax.axis_index("c")
    sid = lax.axis_index("s")
    w = _worker_id()
    r0 = sid * L1_RPT

    def graph_body(g, _):
        pltpu.sync_copy(t_ref.at[pl.ds(g * NP + r0, L1_RPT)],
                        acc.at[pl.ds(r0, L1_RPT)])
        plsc.subcore_barrier()
        row0 = g * (EP1 // CH) + w * (NB1 * KB)
        _mp_worker_loop(t_ref, s2d_ref, d2d_ref, acc, (is0, is1), (id0, id1),
                        (rows0, rows1), (gsem0, gsem1), (isem0, isem1),
                        row0, NB1)
        plsc.subcore_barrier()
        pltpu.sync_copy(acc.at[pl.ds(r0, L1_RPT)],
                        out_ref.at[g, cid, pl.ds(r0, L1_RPT)])
        plsc.subcore_barrier()
        return _

    lax.fori_loop(0, NGG, graph_body, 0)


def _mp1_call(t_stack, src2d, dstraw2d):
    f = pl.kernel(
        _mp1_body,
        compiler_params=_SC_PARAMS,
        out_type=jax.ShapeDtypeStruct((NGG, NC, NP, 128), jnp.float32),
        mesh=_MESH,
        scratch_types=[
            pltpu.VMEM_SHARED((NP, 128), jnp.float32),
            pltpu.VMEM((KB, CH), jnp.int32),
            pltpu.VMEM((KB, CH), jnp.int32),
            pltpu.VMEM((KB, CH), jnp.int32),
            pltpu.VMEM((KB, CH), jnp.int32),
            pltpu.VMEM((CH, 128), jnp.float32),
            pltpu.VMEM((CH, 128), jnp.float32),
            pltpu.SemaphoreType.DMA,
            pltpu.SemaphoreType.DMA,
            pltpu.SemaphoreType.DMA,
            pltpu.SemaphoreType.DMA,
        ],
    )
    return f(t_stack, src2d, dstraw2d)


L2_RPT = NGNP // NS


def _mp2_body(t_ref, s2d_ref, d2d_ref, out_ref, acc, is0, is1, id0, id1,
              rows0, rows1, gsem0, gsem1, isem0, isem1):
    cid = lax.axis_index("c")
    sid = lax.axis_index("s")
    w = _worker_id()
    r0 = sid * L2_RPT
    pltpu.sync_copy(t_ref.at[pl.ds(r0, L2_RPT)], acc.at[pl.ds(r0, L2_RPT)])
    plsc.subcore_barrier()
    _mp_worker_loop(t_ref, s2d_ref, d2d_ref, acc, (is0, is1), (id0, id1),
                    (rows0, rows1), (gsem0, gsem1), (isem0, isem1),
                    w * (NB2 * KB), NB2)
    plsc.subcore_barrier()
    pltpu.sync_copy(acc.at[pl.ds(r0, L2_RPT)],
                    out_ref.at[cid, pl.ds(r0, L2_RPT)])


def _mp2_call(t2_stack, src2d, dstglob2d):
    f = pl.kernel(
        _mp2_body,
        compiler_params=_SC_PARAMS,
        out_type=jax.ShapeDtypeStruct((NC, NGNP, 16), jnp.float32),
        mesh=_MESH,
        scratch_types=[
            pltpu.VMEM_SHARED((NGNP, 16), jnp.float32),
            pltpu.VMEM((KB, CH), jnp.int32),
            pltpu.VMEM((KB, CH), jnp.int32),
            pltpu.VMEM((KB, CH), jnp.int32),
            pltpu.VMEM((KB, CH), jnp.int32),
            pltpu.VMEM((CH, 16), jnp.float32),
            pltpu.VMEM((CH, 16), jnp.float32),
            pltpu.SemaphoreType.DMA,
            pltpu.SemaphoreType.DMA,
            pltpu.SemaphoreType.DMA,
            pltpu.SemaphoreType.DMA,
        ],
    )
    return f(t2_stack, src2d, dstglob2d)


DG_RPT = NGNP // NS


def _deg_body(d2d_ref, zeros_ref, out_ref, acc, id0, id1, ones_v, ssem,
              isem):
    cid = lax.axis_index("c")
    sid = lax.axis_index("s")
    w = _worker_id()
    r0 = sid * DG_RPT
    pltpu.sync_copy(zeros_ref.at[pl.ds(r0, DG_RPT)], acc.at[pl.ds(r0, DG_RPT)])
    for i in range(CH // 16):
        ones_v[pl.ds(16 * i, 16)] = jnp.full((16,), 1.0, jnp.float32)
    plsc.subcore_barrier()
    row0 = w * (NB2 * KB)
    _load_idx_batch(d2d_ref, row0, id0)

    def outer(bb, _):
        for p in (0, 1):
            k = 2 * bb + p
            cur = id0 if p == 0 else id1
            nxt_buf = id1 if p == 0 else id0
            li = pltpu.async_copy(d2d_ref.at[pl.ds(row0 + (k + 1) * KB, KB)],
                                  nxt_buf, isem)
            descs = [pltpu.async_copy(ones_v, acc.at[cur.at[i]], ssem,
                                      add=True) for i in range(KB)]
            for d in descs:
                d.wait()
            li.wait()
        return _

    lax.fori_loop(0, NB2 // 2, outer, 0)
    plsc.subcore_barrier()
    pltpu.sync_copy(acc.at[pl.ds(r0, DG_RPT)],
                    out_ref.at[cid, pl.ds(r0, DG_RPT)])


def _deg_call(dstglob2d):
    f = pl.kernel(
        _deg_body,
        compiler_params=_SC_PARAMS,
        out_type=jax.ShapeDtypeStruct((NC, NGNP), jnp.float32),
        mesh=_MESH,
        scratch_types=[
            pltpu.VMEM_SHARED((NGNP,), jnp.float32),
            pltpu.VMEM((KB, CH), jnp.int32),
            pltpu.VMEM((KB, CH), jnp.int32),
            pltpu.VMEM((CH,), jnp.float32),
            pltpu.SemaphoreType.DMA,
            pltpu.SemaphoreType.DMA,
        ],
    )
    return f(dstglob2d, jnp.zeros((NGNP,), jnp.float32))


def _dec_body(z_ref, u2d_ref, v2d_ref, out_ref, iu0, iv0,
              rows0, rows1, gsem0, gsem1, stsem0, stsem1):
    w = _worker_id()
    rows = (rows0, rows1)
    gsems = (gsem0, gsem1)
    stsems = (stsem0, stsem1)
    row0 = w * (NBD * KB)

    def outer(k, _):
        _load_idx_batch(u2d_ref, row0 + k * KB, iu0)
        _load_idx_batch(v2d_ref, row0 + k * KB, iv0)
        ebase = (row0 + k * KB) * CH
        gu = [None, None]
        gv = [None, None]
        st = [None, None]
        gu[0] = pltpu.async_copy(z_ref.at[iu0.at[0]], rows[0], gsems[0])
        for i in range(KB):
            b = i % 2
            gu[b].wait()
            gv[b] = pltpu.async_copy(z_ref.at[iv0.at[i]], rows[b],
                                     gsems[b], add=True)
            if i + 1 < KB:
                if st[1 - b] is not None:
                    st[1 - b].wait()
                gu[1 - b] = pltpu.async_copy(z_ref.at[iu0.at[i + 1]],
                                             rows[1 - b], gsems[1 - b])
            gv[b].wait()
            st[b] = pltpu.async_copy(
                rows[b], out_ref.at[pl.ds(ebase + i * CH, CH)], stsems[b])
        st[0].wait()
        st[1].wait()
        return _

    lax.fori_loop(0, NBD, outer, 0)


def _dec_call(zcat, u2d, v2d):
    f = pl.kernel(
        _dec_body,
        compiler_params=_SC_PARAMS,
        out_type=jax.ShapeDtypeStruct((EPD, 112), jnp.float32),
        mesh=_MESH,
        scratch_types=[
            pltpu.VMEM((KB, CH), jnp.int32),
            pltpu.VMEM((KB, CH), jnp.int32),
            pltpu.VMEM((CH, 112), jnp.float32),
            pltpu.VMEM((CH, 112), jnp.float32),
            pltpu.SemaphoreType.DMA,
            pltpu.SemaphoreType.DMA,
            pltpu.SemaphoreType.DMA,
            pltpu.SemaphoreType.DMA,
        ],
    )
    return f(zcat, u2d, v2d)


def _decode_mlp_body(emb_ref, wg_ref, bg_ref, ex_ref, l1w_ref, l1b_ref,
                     l2w_ref, l2b_ref, l3w_ref, l3b_ref, out_ref):
    emb = emb_ref[...]
    wv = jnp.dot(emb, wg_ref[...], preferred_element_type=jnp.float32) + bg_ref[...]
    wvx = jnp.dot(wv, ex_ref[...], preferred_element_type=jnp.float32)
    feats = emb * wvx
    h = jnp.maximum(jnp.dot(feats, l1w_ref[...], preferred_element_type=jnp.float32)
                    + l1b_ref[...], 0.0)
    h = jnp.maximum(jnp.dot(h, l2w_ref[...], preferred_element_type=jnp.float32)
                    + l2b_ref[...], 0.0)
    o = jnp.dot(h, l3w_ref[...], preferred_element_type=jnp.float32) + l3b_ref[...]
    out_ref[...] = 1.0 / (1.0 + jnp.exp(-o))


def _decode_mlp(emb, params):
    ne = emb.shape[0]
    d = NGG * 16
    wg = jnp.zeros((d, NGG), jnp.float32)
    for i, g in enumerate(USED_GRAPHS):
        wg = wg.at[16 * i:16 * (i + 1), i].set(params["w_" + g + "_W"][:, 0])
    bg = jnp.stack([params["w_" + g + "_b"][0] for g in USED_GRAPHS])[None, :]
    ex = jnp.repeat(jnp.eye(NGG, dtype=jnp.float32), 16, axis=1)

    grid = ne // DEC_BLOCK
    full = lambda shape: pl.BlockSpec(shape, lambda i: (0, 0))
    out = pl.pallas_call(
        _decode_mlp_body,
        grid=(grid,),
        in_specs=[
            pl.BlockSpec((DEC_BLOCK, d), lambda i: (i, 0)),
            full(wg.shape), full(bg.shape), full(ex.shape),
            full(params["L1_W"].shape), full((1, 32)),
            full(params["L2_W"].shape), full((1, 16)),
            full(params["L3_W"].shape), full((1, 1)),
        ],
        out_specs=pl.BlockSpec((DEC_BLOCK, 1), lambda i: (i, 0)),
        out_shape=jax.ShapeDtypeStruct((ne, 1), jnp.float32),
    )(emb, wg, bg, ex,
      params["L1_W"], params["L1_b"][None, :],
      params["L2_W"], params["L2_b"][None, :],
      params["L3_W"], params["L3_b"][None, :])
    return out[:, 0]


def _pad_reshape_idx(parts):
    flat = jnp.concatenate(parts + [jnp.zeros((BE,), jnp.int32)])
    return flat.reshape(-1, CH)


def kernel(x, sl_pos, sl_neg, kg_ppi, kg_reactome, kg_corum, kg_go_f,
           kg_go_c, kg_go_p, kg_kegg, params):
    edge_map = {"sl": sl_pos, "ppi": kg_ppi, "reactome": kg_reactome,
                "go_f": kg_go_f, "go_c": kg_go_c, "go_p": kg_go_p,
                "kegg": kg_kegg}
    eis = [edge_map[g] for g in USED_GRAPHS]
    padn = EP1 - E
    src2d = _pad_reshape_idx(
        [a for g, ei in enumerate(eis)
         for a in (ei[0] + g * NP, jnp.full((padn,), g * NP + N, jnp.int32))])
    dstraw2d = _pad_reshape_idx(
        [a for ei in eis
         for a in (ei[1], jnp.full((padn,), N, jnp.int32))])
    dstglob2d = _pad_reshape_idx(
        [a for g, ei in enumerate(eis)
         for a in (ei[1] + g * NP, jnp.full((padn,), g * NP + N, jnp.int32))])

    dp = _deg_call(dstglob2d)
    deg = dp[0] + dp[1] + 1.0
    dinv = (deg ** -0.5)[:, None]

    pad = jnp.zeros((NP - N, 128), jnp.float32)
    t1 = jnp.concatenate(
        [jnp.concatenate([x @ params[g + "_W1"], pad])
         for g in USED_GRAPHS], axis=0) * dinv

    p1 = _mp1_call(t1, src2d, dstraw2d)
    s1 = (p1[:, 0] + p1[:, 1]).reshape(NGNP, 128) - t1
    b1 = jnp.concatenate(
        [jnp.broadcast_to(params[g + "_b1"], (NP, 128)) for g in USED_GRAPHS])
    h = jax.nn.relu(dinv * s1 + b1)

    t2 = jnp.concatenate(
        [h[g * NP:(g + 1) * NP] @ params[USED_GRAPHS[g] + "_W2"]
         for g in range(NGG)], axis=0) * dinv

    p2 = _mp2_call(t2, src2d, dstglob2d)
    b2 = jnp.concatenate(
        [jnp.broadcast_to(params[g + "_b2"], (NP, 16)) for g in USED_GRAPHS])
    z = dinv * (p2[0] + p2[1] - t2) + b2
    zcat = z.reshape(NGG, NP, 16)[:, :N].transpose(1, 0, 2).reshape(N, NGG * 16)

    padd = EPD - EDEC
    u2d = _pad_reshape_idx([sl_pos[0], sl_neg[0],
                            jnp.zeros((padd,), jnp.int32)])
    v2d = _pad_reshape_idx([sl_pos[1], sl_neg[1],
                            jnp.zeros((padd,), jnp.int32)])
    emb = _dec_call(zcat, u2d, v2d)
    return _decode_mlp(emb, params)[:EDEC]


# speedup vs baseline: 1.0481x; 1.0481x over previous
import functools

import jax
import jax.numpy as jnp
from jax import lax
from jax.experimental import pallas as pl
from jax.experimental.pallas import tpu as pltpu
from jax.experimental.pallas import tpu_sc as plsc

USED_GRAPHS = ["sl", "ppi", "reactome", "go_f", "go_c", "go_p", "kegg"]
NGG = len(USED_GRAPHS)
N = 10000
NP = 10240
E = 320000
NC, NS = 2, 16
NW = NC * NS
NGNP = NGG * NP

CH = 128
KB = 8
BE = CH * KB
NB1 = 10
NB1_F, NB1_S = 16, 4
EP1 = NW * NB1 * BE
NB2 = NGG * NB1
EDEC = 2 * E
NBD = 20
NBD_F, NBD_S = 30, 10
EPD = NW * NBD * BE

DEC_BLOCK = 2048

_MESH = plsc.VectorSubcoreMesh(core_axis_name="c", subcore_axis_name="s",
                               num_cores=NC, num_subcores=NS)
_SC_PARAMS = pltpu.CompilerParams(use_tc_tiling_on_sc=False)


def _worker_id():
    return lax.axis_index("s") * NC + lax.axis_index("c")


def _load_idx_batch(src2d_ref, row0, dst_buf):
    pltpu.sync_copy(src2d_ref.at[pl.ds(row0, KB)], dst_buf)


def _mp_batch(t_ref, acc, Is, Id, rows, gsems):
    g = [None, None]
    g[0] = pltpu.async_copy(t_ref.at[Is.at[0]], rows[0], gsems[0])
    for i in range(KB):
        b = i % 2
        if i + 1 < KB:
            g[1 - b] = pltpu.async_copy(t_ref.at[Is.at[i + 1]], rows[1 - b],
                                        gsems[1 - b])
        g[b].wait()
        pltpu.sync_copy(rows[b], acc.at[Id.at[i]], add=True)


def _mp_worker_loop(t_ref, s2d_ref, d2d_ref, acc, Is2, Id2, rows, gsems,
                    isems, row0, nb):
    _load_idx_batch(s2d_ref, row0, Is2[0])
    _load_idx_batch(d2d_ref, row0, Id2[0])

    def outer(bb, _):
        for p in (0, 1):
            k = 2 * bb + p
            nxt = row0 + (k + 1) * KB
            li = pltpu.async_copy(s2d_ref.at[pl.ds(nxt, KB)], Is2[1 - p],
                                  isems[0])
            ld = pltpu.async_copy(d2d_ref.at[pl.ds(nxt, KB)], Id2[1 - p],
                                  isems[1])
            _mp_batch(t_ref, acc, Is2[p], Id2[p], rows, gsems)
            li.wait()
            ld.wait()
        return _

    lax.fori_loop(0, nb // 2, outer, 0)


L1_RPT = NP // NS


def _mp1_body(t_ref, s2d_ref, d2d_ref, out_ref, acc, is0, is1, id0, id1,
              rows0, rows1, gsem0, gsem1, isem0, isem1):
    cid = lax.axis_index("c")
    sid = lax.axis_index("s")
    w = _worker_id()
    r0 = sid * L1_RPT

    nb = jnp.where(cid == 0, NB1_F, NB1_S)
    wrow = jnp.where(cid == 0, sid * (NB1_F * KB),
                     NS * (NB1_F * KB) + sid * (NB1_S * KB))

    def graph_body(g, _):
        pltpu.sync_copy(t_ref.at[pl.ds(g * NP + r0, L1_RPT)],
                        acc.at[pl.ds(r0, L1_RPT)])
        plsc.subcore_barrier()
        row0 = g * (EP1 // CH) + wrow
        _mp_worker_loop(t_ref, s2d_ref, d2d_ref, acc, (is0, is1), (id0, id1),
                        (rows0, rows1), (gsem0, gsem1), (isem0, isem1),
                        row0, nb)
        plsc.subcore_barrier()
        pltpu.sync_copy(acc.at[pl.ds(r0, L1_RPT)],
                        out_ref.at[g, cid, pl.ds(r0, L1_RPT)])
        plsc.subcore_barrier()
        return _

    lax.fori_loop(0, NGG, graph_body, 0)


def _mp1_call(t_stack, src2d, dstraw2d):
    f = pl.kernel(
        _mp1_body,
        compiler_params=_SC_PARAMS,
        out_type=jax.ShapeDtypeStruct((NGG, NC, NP, 128), jnp.float32),
        mesh=_MESH,
        scratch_types=[
            pltpu.VMEM_SHARED((NP, 128), jnp.float32),
            pltpu.VMEM((KB, CH), jnp.int32),
            pltpu.VMEM((KB, CH), jnp.int32),
            pltpu.VMEM((KB, CH), jnp.int32),
            pltpu.VMEM((KB, CH), jnp.int32),
            pltpu.VMEM((CH, 128), jnp.float32),
            pltpu.VMEM((CH, 128), jnp.float32),
            pltpu.SemaphoreType.DMA,
            pltpu.SemaphoreType.DMA,
            pltpu.SemaphoreType.DMA,
            pltpu.SemaphoreType.DMA,
        ],
    )
    return f(t_stack, src2d, dstraw2d)


L2_RPT = NGNP // NS


def _mp2_body(t_ref, s2d_ref, d2d_ref, out_ref, acc, is0, is1, id0, id1,
              rows0, rows1, gsem0, gsem1, isem0, isem1):
    cid = lax.axis_index("c")
    sid = lax.axis_index("s")
    w = _worker_id()
    r0 = sid * L2_RPT
    pltpu.sync_copy(t_ref.at[pl.ds(r0, L2_RPT)], acc.at[pl.ds(r0, L2_RPT)])
    plsc.subcore_barrier()
    _mp_worker_loop(t_ref, s2d_ref, d2d_ref, acc, (is0, is1), (id0, id1),
                    (rows0, rows1), (gsem0, gsem1), (isem0, isem1),
                    w * (NB2 * KB), NB2)
    plsc.subcore_barrier()
    pltpu.sync_copy(acc.at[pl.ds(r0, L2_RPT)],
                    out_ref.at[cid, pl.ds(r0, L2_RPT)])


def _mp2_call(t2_stack, src2d, dstglob2d):
    f = pl.kernel(
        _mp2_body,
        compiler_params=_SC_PARAMS,
        out_type=jax.ShapeDtypeStruct((NC, NGNP, 16), jnp.float32),
        mesh=_MESH,
        scratch_types=[
            pltpu.VMEM_SHARED((NGNP, 16), jnp.float32),
            pltpu.VMEM((KB, CH), jnp.int32),
            pltpu.VMEM((KB, CH), jnp.int32),
            pltpu.VMEM((KB, CH), jnp.int32),
            pltpu.VMEM((KB, CH), jnp.int32),
            pltpu.VMEM((CH, 16), jnp.float32),
            pltpu.VMEM((CH, 16), jnp.float32),
            pltpu.SemaphoreType.DMA,
            pltpu.SemaphoreType.DMA,
            pltpu.SemaphoreType.DMA,
            pltpu.SemaphoreType.DMA,
        ],
    )
    return f(t2_stack, src2d, dstglob2d)


DG_RPT = NGNP // NS


def _deg_body(d2d_ref, zeros_ref, out_ref, acc, id0, id1, ones_v, ssem,
              isem):
    cid = lax.axis_index("c")
    sid = lax.axis_index("s")
    w = _worker_id()
    r0 = sid * DG_RPT
    pltpu.sync_copy(zeros_ref.at[pl.ds(r0, DG_RPT)], acc.at[pl.ds(r0, DG_RPT)])
    for i in range(CH // 16):
        ones_v[pl.ds(16 * i, 16)] = jnp.full((16,), 1.0, jnp.float32)
    plsc.subcore_barrier()
    row0 = w * (NB2 * KB)
    _load_idx_batch(d2d_ref, row0, id0)

    def outer(bb, _):
        for p in (0, 1):
            k = 2 * bb + p
            cur = id0 if p == 0 else id1
            nxt_buf = id1 if p == 0 else id0
            li = pltpu.async_copy(d2d_ref.at[pl.ds(row0 + (k + 1) * KB, KB)],
                                  nxt_buf, isem)
            descs = [pltpu.async_copy(ones_v, acc.at[cur.at[i]], ssem,
                                      add=True) for i in range(KB)]
            for d in descs:
                d.wait()
            li.wait()
        return _

    lax.fori_loop(0, NB2 // 2, outer, 0)
    plsc.subcore_barrier()
    pltpu.sync_copy(acc.at[pl.ds(r0, DG_RPT)],
                    out_ref.at[cid, pl.ds(r0, DG_RPT)])


def _deg_call(dstglob2d):
    f = pl.kernel(
        _deg_body,
        compiler_params=_SC_PARAMS,
        out_type=jax.ShapeDtypeStruct((NC, NGNP), jnp.float32),
        mesh=_MESH,
        scratch_types=[
            pltpu.VMEM_SHARED((NGNP,), jnp.float32),
            pltpu.VMEM((KB, CH), jnp.int32),
            pltpu.VMEM((KB, CH), jnp.int32),
            pltpu.VMEM((CH,), jnp.float32),
            pltpu.SemaphoreType.DMA,
            pltpu.SemaphoreType.DMA,
        ],
    )
    return f(dstglob2d, jnp.zeros((NGNP,), jnp.float32))


def _dec_body(z_ref, u2d_ref, v2d_ref, out_ref, iu0, iv0,
              rows0, rows1, gsem0, gsem1, stsem0, stsem1):
    cid = lax.axis_index("c")
    sid = lax.axis_index("s")
    rows = (rows0, rows1)
    gsems = (gsem0, gsem1)
    stsems = (stsem0, stsem1)
    nbd = jnp.where(cid == 0, NBD_F, NBD_S)
    row0 = jnp.where(cid == 0, sid * (NBD_F * KB),
                     NS * (NBD_F * KB) + sid * (NBD_S * KB))

    def outer(k, _):
        _load_idx_batch(u2d_ref, row0 + k * KB, iu0)
        _load_idx_batch(v2d_ref, row0 + k * KB, iv0)
        ebase = (row0 + k * KB) * CH
        gu = [None, None]
        gv = [None, None]
        st = [None, None]
        gu[0] = pltpu.async_copy(z_ref.at[iu0.at[0]], rows[0], gsems[0])
        for i in range(KB):
            b = i % 2
            gu[b].wait()
            gv[b] = pltpu.async_copy(z_ref.at[iv0.at[i]], rows[b],
                                     gsems[b], add=True)
            if i + 1 < KB:
                if st[1 - b] is not None:
                    st[1 - b].wait()
                gu[1 - b] = pltpu.async_copy(z_ref.at[iu0.at[i + 1]],
                                             rows[1 - b], gsems[1 - b])
            gv[b].wait()
            st[b] = pltpu.async_copy(
                rows[b], out_ref.at[pl.ds(ebase + i * CH, CH)], stsems[b])
        st[0].wait()
        st[1].wait()
        return _

    lax.fori_loop(0, nbd, outer, 0)


def _dec_call(zcat, u2d, v2d):
    f = pl.kernel(
        _dec_body,
        compiler_params=_SC_PARAMS,
        out_type=jax.ShapeDtypeStruct((EPD, 112), jnp.float32),
        mesh=_MESH,
        scratch_types=[
            pltpu.VMEM((KB, CH), jnp.int32),
            pltpu.VMEM((KB, CH), jnp.int32),
            pltpu.VMEM((CH, 112), jnp.float32),
            pltpu.VMEM((CH, 112), jnp.float32),
            pltpu.SemaphoreType.DMA,
            pltpu.SemaphoreType.DMA,
            pltpu.SemaphoreType.DMA,
            pltpu.SemaphoreType.DMA,
        ],
    )
    return f(zcat, u2d, v2d)


def _decode_mlp_body(emb_ref, wg_ref, bg_ref, ex_ref, l1w_ref, l1b_ref,
                     l2w_ref, l2b_ref, l3w_ref, l3b_ref, out_ref):
    emb = emb_ref[...]
    wv = jnp.dot(emb, wg_ref[...], preferred_element_type=jnp.float32) + bg_ref[...]
    wvx = jnp.dot(wv, ex_ref[...], preferred_element_type=jnp.float32)
    feats = emb * wvx
    h = jnp.maximum(jnp.dot(feats, l1w_ref[...], preferred_element_type=jnp.float32)
                    + l1b_ref[...], 0.0)
    h = jnp.maximum(jnp.dot(h, l2w_ref[...], preferred_element_type=jnp.float32)
                    + l2b_ref[...], 0.0)
    o = jnp.dot(h, l3w_ref[...], preferred_element_type=jnp.float32) + l3b_ref[...]
    out_ref[...] = 1.0 / (1.0 + jnp.exp(-o))


def _decode_mlp(emb, params):
    ne = emb.shape[0]
    d = NGG * 16
    wg = jnp.zeros((d, NGG), jnp.float32)
    for i, g in enumerate(USED_GRAPHS):
        wg = wg.at[16 * i:16 * (i + 1), i].set(params["w_" + g + "_W"][:, 0])
    bg = jnp.stack([params["w_" + g + "_b"][0] for g in USED_GRAPHS])[None, :]
    ex = jnp.repeat(jnp.eye(NGG, dtype=jnp.float32), 16, axis=1)

    grid = ne // DEC_BLOCK
    full = lambda shape: pl.BlockSpec(shape, lambda i: (0, 0))
    out = pl.pallas_call(
        _decode_mlp_body,
        grid=(grid,),
        in_specs=[
            pl.BlockSpec((DEC_BLOCK, d), lambda i: (i, 0)),
            full(wg.shape), full(bg.shape), full(ex.shape),
            full(params["L1_W"].shape), full((1, 32)),
            full(params["L2_W"].shape), full((1, 16)),
            full(params["L3_W"].shape), full((1, 1)),
        ],
        out_specs=pl.BlockSpec((DEC_BLOCK, 1), lambda i: (i, 0)),
        out_shape=jax.ShapeDtypeStruct((ne, 1), jnp.float32),
    )(emb, wg, bg, ex,
      params["L1_W"], params["L1_b"][None, :],
      params["L2_W"], params["L2_b"][None, :],
      params["L3_W"], params["L3_b"][None, :])
    return out[:, 0]


def _pad_reshape_idx(parts):
    flat = jnp.concatenate(parts + [jnp.zeros((BE,), jnp.int32)])
    return flat.reshape(-1, CH)


def kernel(x, sl_pos, sl_neg, kg_ppi, kg_reactome, kg_corum, kg_go_f,
           kg_go_c, kg_go_p, kg_kegg, params):
    edge_map = {"sl": sl_pos, "ppi": kg_ppi, "reactome": kg_reactome,
                "go_f": kg_go_f, "go_c": kg_go_c, "go_p": kg_go_p,
                "kegg": kg_kegg}
    eis = [edge_map[g] for g in USED_GRAPHS]
    padn = EP1 - E
    src2d = _pad_reshape_idx(
        [a for g, ei in enumerate(eis)
         for a in (ei[0] + g * NP, jnp.full((padn,), g * NP + N, jnp.int32))])
    dstraw2d = _pad_reshape_idx(
        [a for ei in eis
         for a in (ei[1], jnp.full((padn,), N, jnp.int32))])
    dstglob2d = _pad_reshape_idx(
        [a for g, ei in enumerate(eis)
         for a in (ei[1] + g * NP, jnp.full((padn,), g * NP + N, jnp.int32))])

    dp = _deg_call(dstglob2d)
    deg = dp[0] + dp[1] + 1.0
    dinv = (deg ** -0.5)[:, None]

    pad = jnp.zeros((NP - N, 128), jnp.float32)
    t1 = jnp.concatenate(
        [jnp.concatenate([x @ params[g + "_W1"], pad])
         for g in USED_GRAPHS], axis=0) * dinv

    p1 = _mp1_call(t1, src2d, dstraw2d)
    s1 = (p1[:, 0] + p1[:, 1]).reshape(NGNP, 128) - t1
    b1 = jnp.concatenate(
        [jnp.broadcast_to(params[g + "_b1"], (NP, 128)) for g in USED_GRAPHS])
    h = jax.nn.relu(dinv * s1 + b1)

    t2 = jnp.concatenate(
        [h[g * NP:(g + 1) * NP] @ params[USED_GRAPHS[g] + "_W2"]
         for g in range(NGG)], axis=0) * dinv

    p2 = _mp2_call(t2, src2d, dstglob2d)
    b2 = jnp.concatenate(
        [jnp.broadcast_to(params[g + "_b2"], (NP, 16)) for g in USED_GRAPHS])
    z = dinv * (p2[0] + p2[1] - t2) + b2
    zcat = z.reshape(NGG, NP, 16)[:, :N].transpose(1, 0, 2).reshape(N, NGG * 16)

    padd = EPD - EDEC
    u2d = _pad_reshape_idx([sl_pos[0], sl_neg[0],
                            jnp.zeros((padd,), jnp.int32)])
    v2d = _pad_reshape_idx([sl_pos[1], sl_neg[1],
                            jnp.zeros((padd,), jnp.int32)])
    emb = _dec_call(zcat, u2d, v2d)
    return _decode_mlp(emb, params)[:EDEC]
